# Initial kernel scaffold; baseline (speedup 1.0000x reference)
import jax
import jax.numpy as jnp
from jax.experimental import pallas as pl


def kernel(x, edge_index_0, edge_index_1, W0, b0, Wg0, bg0, W1, b1, Wg1, bg1):
    raise NotImplementedError("write your pallas kernel here")


# speedup vs baseline: 8.1374x; 8.1374x over previous
import functools

import jax
import jax.numpy as jnp
from jax import lax
from jax.experimental import pallas as pl
from jax.experimental.pallas import tpu as pltpu
from jax.experimental.pallas import tpu_sc as plsc

N = 10000
E = 320000
D = 128
NPAD = 10240

NC = 2
NS = 16
EPT = E // (NC * NS)
CH = 128
NFULL = EPT // CH
PAIRS = NFULL // 2
TAIL = EPT - NFULL * CH
RPT = NPAD // NS

RBLK = 400


def _mm_gate_body(x_ref, wc_ref, bg_ref, sup_ref, g_ref):
    y = jnp.dot(x_ref[...], wc_ref[...], preferred_element_type=jnp.float32)
    sup_ref[...] = y[:, :D]
    g_ref[...] = jax.nn.sigmoid(y[:, D:] + bg_ref[...])


def _mm_gate(x, wc, bg):
    grid = N // RBLK
    return pl.pallas_call(
        _mm_gate_body,
        grid=(grid,),
        in_specs=[
            pl.BlockSpec((RBLK, D), lambda i: (i, 0)),
            pl.BlockSpec((D, 2 * D), lambda i: (0, 0)),
            pl.BlockSpec((1, D), lambda i: (0, 0)),
        ],
        out_specs=[
            pl.BlockSpec((RBLK, D), lambda i: (i, 0)),
            pl.BlockSpec((RBLK, D), lambda i: (i, 0)),
        ],
        out_shape=[
            jax.ShapeDtypeStruct((N, D), jnp.float32),
            jax.ShapeDtypeStruct((N, D), jnp.float32),
        ],
    )(x, wc, bg)


def _combine_mm_body(agg_ref, g_ref, x_ref, b_ref, wc_ref, bg_ref,
                     sup_ref, g1_ref):
    h = agg_ref[0] + agg_ref[1] + b_ref[...]
    g = g_ref[...]
    out0 = g * h + (1.0 - g) * x_ref[...]
    y = jnp.dot(out0, wc_ref[...], preferred_element_type=jnp.float32)
    sup_ref[...] = y[:, :D]
    g1_ref[...] = jax.nn.sigmoid(y[:, D:] + bg_ref[...])


def _combine_mm(agg2, g, x, b, wc, bg):
    grid = N // RBLK
    return pl.pallas_call(
        _combine_mm_body,
        grid=(grid,),
        in_specs=[
            pl.BlockSpec((2, RBLK, D), lambda i: (0, i, 0)),
            pl.BlockSpec((RBLK, D), lambda i: (i, 0)),
            pl.BlockSpec((RBLK, D), lambda i: (i, 0)),
            pl.BlockSpec((1, D), lambda i: (0, 0)),
            pl.BlockSpec((D, 2 * D), lambda i: (0, 0)),
            pl.BlockSpec((1, D), lambda i: (0, 0)),
        ],
        out_specs=[
            pl.BlockSpec((RBLK, D), lambda i: (i, 0)),
            pl.BlockSpec((RBLK, D), lambda i: (i, 0)),
        ],
        out_shape=[
            jax.ShapeDtypeStruct((N, D), jnp.float32),
            jax.ShapeDtypeStruct((N, D), jnp.float32),
        ],
    )(agg2, g, x, b, wc, bg)


def _combine_final_body(agg_ref, g_ref, x_ref, b_ref, out_ref):
    h = agg_ref[0] + agg_ref[1] + b_ref[...]
    g = g_ref[...]
    out_ref[...] = g * h + (1.0 - g) * x_ref[...]


def _combine_final(agg2, g, x, b):
    grid = N // RBLK
    return pl.pallas_call(
        _combine_final_body,
        grid=(grid,),
        in_specs=[
            pl.BlockSpec((2, RBLK, D), lambda i: (0, i, 0)),
            pl.BlockSpec((RBLK, D), lambda i: (i, 0)),
            pl.BlockSpec((RBLK, D), lambda i: (i, 0)),
            pl.BlockSpec((1, D), lambda i: (0, 0)),
        ],
        out_specs=pl.BlockSpec((RBLK, D), lambda i: (i, 0)),
        out_shape=jax.ShapeDtypeStruct((N, D), jnp.float32),
    )(agg2, g, x, b)


def _sc_agg_body(sup_hbm, src_hbm, dst_hbm, out_hbm,
                 sidx_a, didx_a, rows_a, sidx_b, didx_b, rows_b,
                 sidx_t, didx_t, rows_t, acc, sem_a, sem_b, sem_t):
    cid = lax.axis_index("c")
    sid = lax.axis_index("s")
    ebase = (cid * NS + sid) * EPT

    def _zrow(i, _):
        r = i // (D // 16)
        c = lax.rem(i, D // 16)
        rows_a[r, pl.ds(c * 16, 16)] = jnp.zeros((16,), jnp.float32)
        return 0
    lax.fori_loop(0, CH * (D // 16), _zrow, 0)
    zbase = sid * RPT
    for j in range(RPT // CH):
        pltpu.sync_copy(rows_a, acc.at[pl.ds(zbase + j * CH, CH)])
    plsc.subcore_barrier()

    def _fire(sbuf, rbuf, sem, base):
        pltpu.sync_copy(src_hbm.at[pl.ds(base, CH)], sbuf)
        pltpu.async_copy(sup_hbm.at[sbuf], rbuf, sem)

    def _drain_scatter(sbuf, dbuf, rbuf, sem, base):
        pltpu.make_async_copy(sup_hbm.at[sbuf], rbuf, sem).wait()
        pltpu.sync_copy(dst_hbm.at[pl.ds(base, CH)], dbuf)
        pltpu.sync_copy(rbuf, acc.at[dbuf], add=True)

    _fire(sidx_a, rows_a, sem_a, ebase)

    def _pair(p, _):
        base_a = ebase + (2 * p) * CH
        base_b = base_a + CH
        _fire(sidx_b, rows_b, sem_b, base_b)
        _drain_scatter(sidx_a, didx_a, rows_a, sem_a, base_a)

        @pl.when(p < PAIRS - 1)
        def _():
            _fire(sidx_a, rows_a, sem_a, base_b + CH)

        _drain_scatter(sidx_b, didx_b, rows_b, sem_b, base_b)
        return 0
    lax.fori_loop(0, PAIRS, _pair, 0)

    tbase = ebase + NFULL * CH
    pltpu.sync_copy(src_hbm.at[pl.ds(tbase, TAIL)], sidx_t)
    pltpu.async_copy(sup_hbm.at[sidx_t], rows_t, sem_t).wait()
    pltpu.sync_copy(dst_hbm.at[pl.ds(tbase, TAIL)], didx_t)
    pltpu.sync_copy(rows_t, acc.at[didx_t], add=True)

    plsc.subcore_barrier()
    obase = sid * RPT
    pltpu.sync_copy(acc.at[pl.ds(obase, RPT)],
                    out_hbm.at[cid, pl.ds(obase, RPT)])


@functools.cache
def _sc_agg_kernel():
    return pl.kernel(
        _sc_agg_body,
        out_type=jax.ShapeDtypeStruct((NC, NPAD, D), jnp.float32),
        mesh=plsc.VectorSubcoreMesh(core_axis_name="c", subcore_axis_name="s",
                                    num_cores=NC, num_subcores=NS),
        scratch_types=[
            pltpu.VMEM((CH,), jnp.int32),
            pltpu.VMEM((CH,), jnp.int32),
            pltpu.VMEM((CH, D), jnp.float32),
            pltpu.VMEM((CH,), jnp.int32),
            pltpu.VMEM((CH,), jnp.int32),
            pltpu.VMEM((CH, D), jnp.float32),
            pltpu.VMEM((TAIL,), jnp.int32),
            pltpu.VMEM((TAIL,), jnp.int32),
            pltpu.VMEM((TAIL, D), jnp.float32),
            pltpu.VMEM_SHARED((NPAD, D), jnp.float32),
            pltpu.SemaphoreType.DMA,
            pltpu.SemaphoreType.DMA,
            pltpu.SemaphoreType.DMA,
        ],
    )


def _sc_agg(sup, src, dst):
    return _sc_agg_kernel()(sup, src, dst)


def kernel(x, edge_index_0, edge_index_1, W0, b0, Wg0, bg0, W1, b1, Wg1, bg1):
    assert x.shape == (N, D) and edge_index_0.shape == (2, E)

    wc0 = jnp.concatenate([W0, Wg0], axis=1)
    wc1 = jnp.concatenate([W1, Wg1], axis=1)
    b0r = b0.reshape(1, D)
    bg0r = bg0.reshape(1, D)
    b1r = b1.reshape(1, D)
    bg1r = bg1.reshape(1, D)
    src0, dst0 = edge_index_0[0], edge_index_0[1]
    src1, dst1 = edge_index_1[0], edge_index_1[1]

    sup0, g0 = _mm_gate(x, wc0, bg0r)
    agg0 = _sc_agg(sup0, src0, dst0)
    sup1, g1 = _combine_mm(agg0, g0, x, b0r, wc1, bg1r)
    agg1 = _sc_agg(sup1, src1, dst1)
    return _combine_final(agg1, g1, x, b1r)
